# baseline (speedup 1.0000x reference)
import functools

import jax
import jax.numpy as jnp
from jax import lax
from jax.experimental import pallas as pl
from jax.experimental.pallas import tpu as pltpu, tpu_sc as plsc

N_NODES = 10000
N_EDGES = 320000
NC, NS = 2, 16
NW = NC * NS
EPW = N_EDGES // NW
K = 128
CHUNK = 20
NCHUNK = 4
PAD = NCHUNK * CHUNK * K - EPW
NJUNK = 8
ACC_ROWS = N_NODES + NJUNK
RPT = 624
TAIL = N_NODES - NS * RPT
TAIL_OFF = NS * RPT
FB = 104
NFB = RPT // FB
HIST_W = 16

_MESH = plsc.VectorSubcoreMesh(core_axis_name="c", subcore_axis_name="s")


@functools.partial(
    pl.kernel,
    out_type=jax.ShapeDtypeStruct((NC, N_NODES, HIST_W), jnp.float32),
    mesh=_MESH,
    scratch_types=[
        pltpu.VMEM((CHUNK, K), jnp.int32),
        pltpu.VMEM((K, HIST_W), jnp.float32),
        pltpu.VMEM((FB, HIST_W), jnp.float32),
        pltpu.VMEM_SHARED((ACC_ROWS, HIST_W), jnp.float32),
    ],
    compiler_params=pltpu.CompilerParams(use_tc_tiling_on_sc=False),
)
def _deg_kernel(dst_hbm, out_hbm, dst_v, ones_v, stage_v, hist_sh):
    c = lax.axis_index("c")
    s = lax.axis_index("s")
    wid = s * NC + c

    def fill_stage(i, _):
        stage_v[i] = jnp.zeros((HIST_W,), jnp.float32)
        return 0

    lax.fori_loop(0, FB, fill_stage, 0)

    def fill_ones(i, _):
        ones_v[i] = jnp.ones((HIST_W,), jnp.float32)
        return 0

    lax.fori_loop(0, K, fill_ones, 0)

    def zero_block(k, _):
        pltpu.sync_copy(stage_v, hist_sh.at[pl.ds(s * RPT + k * FB, FB)])
        return 0

    lax.fori_loop(0, NFB, zero_block, 0)

    @pl.when(s == 0)
    def _():
        pltpu.sync_copy(stage_v.at[pl.ds(0, TAIL)],
                        hist_sh.at[pl.ds(TAIL_OFF, TAIL)])

    plsc.subcore_barrier()

    def chunk_loop(ci, _):
        pltpu.sync_copy(dst_hbm.at[wid, ci], dst_v)

        def step(j, _):
            pltpu.sync_copy(ones_v, hist_sh.at[dst_v.at[j]], add=True)
            return 0

        lax.fori_loop(0, CHUNK, step, 0)
        return 0

    lax.fori_loop(0, NCHUNK, chunk_loop, 0)
    plsc.subcore_barrier()

    def flush_block(k, _):
        pltpu.sync_copy(hist_sh.at[pl.ds(s * RPT + k * FB, FB)], stage_v)
        pltpu.sync_copy(stage_v, out_hbm.at[c, pl.ds(s * RPT + k * FB, FB)])
        return 0

    lax.fori_loop(0, NFB, flush_block, 0)

    @pl.when(s == 0)
    def _():
        pltpu.sync_copy(hist_sh.at[pl.ds(TAIL_OFF, TAIL)],
                        stage_v.at[pl.ds(0, TAIL)])
        pltpu.sync_copy(stage_v.at[pl.ds(0, TAIL)],
                        out_hbm.at[c, pl.ds(TAIL_OFF, TAIL)])


def _make_agg_kernel(d):
    @functools.partial(
        pl.kernel,
        out_type=jax.ShapeDtypeStruct((NC, N_NODES, d), jnp.float32),
        mesh=_MESH,
        scratch_types=[
            pltpu.VMEM((CHUNK, K), jnp.int32),
            pltpu.VMEM((CHUNK, K), jnp.int32),
            pltpu.VMEM((2, K, d), jnp.float32),
            pltpu.VMEM_SHARED((ACC_ROWS, d), jnp.float32),
            pltpu.SemaphoreType.DMA,
        ],
        compiler_params=pltpu.CompilerParams(use_tc_tiling_on_sc=False),
    )
    def agg(g_hbm, src_hbm, dst_hbm, out_hbm,
            src_v, dst_v, rows_v, acc_sh, sem):
        c = lax.axis_index("c")
        s = lax.axis_index("s")
        wid = s * NC + c

        sl = pl.ds(s * RPT, RPT)
        pltpu.sync_copy(g_hbm.at[sl], acc_sh.at[sl])

        @pl.when(s == 0)
        def _():
            tl = pl.ds(TAIL_OFF, TAIL)
            pltpu.sync_copy(g_hbm.at[tl], acc_sh.at[tl])

        plsc.subcore_barrier()

        def chunk_loop(ci, _):
            pltpu.sync_copy(src_hbm.at[wid, ci], src_v)
            pltpu.sync_copy(dst_hbm.at[wid, ci], dst_v)
            pltpu.async_copy(g_hbm.at[src_v.at[0]], rows_v.at[0], sem)

            def step(j, _):
                p = j % 2
                pltpu.make_async_copy(
                    g_hbm.at[src_v.at[j]], rows_v.at[p], sem).wait()

                @pl.when(j + 1 < CHUNK)
                def _():
                    pltpu.async_copy(
                        g_hbm.at[src_v.at[j + 1]], rows_v.at[1 - p], sem)

                pltpu.sync_copy(rows_v.at[p], acc_sh.at[dst_v.at[j]], add=True)
                return 0

            lax.fori_loop(0, CHUNK, step, 0)
            return 0

        lax.fori_loop(0, NCHUNK, chunk_loop, 0)
        plsc.subcore_barrier()

        pltpu.sync_copy(acc_sh.at[sl], out_hbm.at[c, sl])

        @pl.when(s == 0)
        def _():
            tl = pl.ds(TAIL_OFF, TAIL)
            pltpu.sync_copy(acc_sh.at[tl], out_hbm.at[c, tl])

    return agg


_agg128 = _make_agg_kernel(128)
_agg64 = _make_agg_kernel(64)


_RB = 1000
_GRID = N_NODES // _RB


def _dis_block(degp_ref):
    deg = degp_ref[0][:, 0:1] + degp_ref[1][:, 0:1] + 1.0
    return lax.rsqrt(deg)


def _tc1_body(degp_ref, x_ref, w1_ref, g1_ref):
    dis = _dis_block(degp_ref)
    g1_ref[...] = dis * jnp.dot(x_ref[...], w1_ref[...],
                                preferred_element_type=jnp.float32)


def _tc2_body(degp_ref, a1_ref, g1_ref, b1_ref, w2_ref, g2_ref):
    dis = _dis_block(degp_ref)
    h = a1_ref[0] + a1_ref[1] - g1_ref[...]
    h = jnp.maximum(dis * h + b1_ref[...], 0.0)
    g2_ref[...] = dis * jnp.dot(h, w2_ref[...],
                                preferred_element_type=jnp.float32)


def _tc3_body(degp_ref, a2_ref, g2_ref, b2_ref, out_ref):
    dis = _dis_block(degp_ref)
    o = a2_ref[0] + a2_ref[1] - g2_ref[...]
    out_ref[...] = jnp.maximum(dis * o + b2_ref[...], 0.0)


def _degp_spec():
    return pl.BlockSpec((NC, _RB, HIST_W), lambda i: (0, i, 0))


def _rows_spec(d):
    return pl.BlockSpec((_RB, d), lambda i: (i, 0))


def _parts_spec(d):
    return pl.BlockSpec((NC, _RB, d), lambda i: (0, i, 0))


def _full_spec(a, b):
    return pl.BlockSpec((a, b), lambda i: (0, 0))


def kernel(x, edge_index, W1, b1, W2, b2):
    ei = edge_index.astype(jnp.int32)
    pad_src = jnp.zeros((NW, PAD), jnp.int32)
    pad_dst = jnp.broadcast_to(
        N_NODES + (jnp.arange(PAD, dtype=jnp.int32) % NJUNK), (NW, PAD))
    src = jnp.concatenate([ei[0].reshape(NW, EPW), pad_src],
                          axis=1).reshape(NW, NCHUNK, CHUNK, K)
    dst = jnp.concatenate([ei[1].reshape(NW, EPW), pad_dst],
                          axis=1).reshape(NW, NCHUNK, CHUNK, K)

    degp = _deg_kernel(dst)

    g1 = pl.pallas_call(
        _tc1_body,
        grid=(_GRID,),
        in_specs=[_degp_spec(), _rows_spec(128), _full_spec(128, 128)],
        out_specs=_rows_spec(128),
        out_shape=jax.ShapeDtypeStruct((N_NODES, 128), jnp.float32),
    )(degp, x, W1)

    a1 = _agg128(g1, src, dst)

    g2 = pl.pallas_call(
        _tc2_body,
        grid=(_GRID,),
        in_specs=[_degp_spec(), _parts_spec(128), _rows_spec(128),
                  _full_spec(1, 128), _full_spec(128, 64)],
        out_specs=_rows_spec(64),
        out_shape=jax.ShapeDtypeStruct((N_NODES, 64), jnp.float32),
    )(degp, a1, g1, b1.reshape(1, 128), W2)

    a2 = _agg64(g2, src, dst)

    out = pl.pallas_call(
        _tc3_body,
        grid=(_GRID,),
        in_specs=[_degp_spec(), _parts_spec(64), _rows_spec(64),
                  _full_spec(1, 64)],
        out_specs=_rows_spec(64),
        out_shape=jax.ShapeDtypeStruct((N_NODES, 64), jnp.float32),
    )(degp, a2, g2, b2.reshape(1, 64))

    return out


# speedup vs baseline: 1.9227x; 1.9227x over previous
import functools

import jax
import jax.numpy as jnp
from jax import lax
from jax.experimental import pallas as pl
from jax.experimental.pallas import tpu as pltpu, tpu_sc as plsc

N_NODES = 10000
N_EDGES = 320000
NC, NS = 2, 16
NW = NC * NS
EPW = N_EDGES // NW
K = 80
CHUNK = 25
NCHUNK = 5
PAD = NCHUNK * CHUNK * K - EPW
NJUNK = 8
ACC_ROWS = N_NODES + NJUNK
RPT = 624
TAIL = N_NODES - NS * RPT
TAIL_OFF = NS * RPT
FB = 104
NFB = RPT // FB
HIST_W = 16

_MESH = plsc.VectorSubcoreMesh(core_axis_name="c", subcore_axis_name="s")


@functools.partial(
    pl.kernel,
    out_type=jax.ShapeDtypeStruct((NC, N_NODES, HIST_W), jnp.float32),
    mesh=_MESH,
    scratch_types=[
        pltpu.VMEM((CHUNK, K), jnp.int32),
        pltpu.VMEM((K, HIST_W), jnp.float32),
        pltpu.VMEM((FB, HIST_W), jnp.float32),
        pltpu.VMEM_SHARED((ACC_ROWS, HIST_W), jnp.float32),
    ],
    compiler_params=pltpu.CompilerParams(use_tc_tiling_on_sc=False),
)
def _deg_kernel(dst_hbm, out_hbm, dst_v, ones_v, stage_v, hist_sh):
    c = lax.axis_index("c")
    s = lax.axis_index("s")
    wid = s * NC + c

    def fill_stage(i, _):
        stage_v[i] = jnp.zeros((HIST_W,), jnp.float32)
        return 0

    lax.fori_loop(0, FB, fill_stage, 0)

    def fill_ones(i, _):
        ones_v[i] = jnp.ones((HIST_W,), jnp.float32)
        return 0

    lax.fori_loop(0, K, fill_ones, 0)

    def zero_block(k, _):
        pltpu.sync_copy(stage_v, hist_sh.at[pl.ds(s * RPT + k * FB, FB)])
        return 0

    lax.fori_loop(0, NFB, zero_block, 0)

    @pl.when(s == 0)
    def _():
        pltpu.sync_copy(stage_v.at[pl.ds(0, TAIL)],
                        hist_sh.at[pl.ds(TAIL_OFF, TAIL)])

    plsc.subcore_barrier()

    def chunk_loop(ci, _):
        pltpu.sync_copy(dst_hbm.at[wid, ci], dst_v)

        def step(j, _):
            pltpu.sync_copy(ones_v, hist_sh.at[dst_v.at[j]], add=True)
            return 0

        lax.fori_loop(0, CHUNK, step, 0)
        return 0

    lax.fori_loop(0, NCHUNK, chunk_loop, 0)
    plsc.subcore_barrier()

    def flush_block(k, _):
        pltpu.sync_copy(hist_sh.at[pl.ds(s * RPT + k * FB, FB)], stage_v)
        pltpu.sync_copy(stage_v, out_hbm.at[c, pl.ds(s * RPT + k * FB, FB)])
        return 0

    lax.fori_loop(0, NFB, flush_block, 0)

    @pl.when(s == 0)
    def _():
        pltpu.sync_copy(hist_sh.at[pl.ds(TAIL_OFF, TAIL)],
                        stage_v.at[pl.ds(0, TAIL)])
        pltpu.sync_copy(stage_v.at[pl.ds(0, TAIL)],
                        out_hbm.at[c, pl.ds(TAIL_OFF, TAIL)])


def _make_agg_kernel(d):
    @functools.partial(
        pl.kernel,
        out_type=jax.ShapeDtypeStruct((NC, N_NODES, d), jnp.float32),
        mesh=_MESH,
        scratch_types=[
            pltpu.VMEM((CHUNK, K), jnp.int32),
            pltpu.VMEM((CHUNK, K), jnp.int32),
            pltpu.VMEM((2, K, d), jnp.float32),
            pltpu.VMEM_SHARED((ACC_ROWS, d), jnp.float32),
            pltpu.SemaphoreType.DMA,
        ],
        compiler_params=pltpu.CompilerParams(use_tc_tiling_on_sc=False),
    )
    def agg(g_hbm, src_hbm, dst_hbm, out_hbm,
            src_v, dst_v, rows_v, acc_sh, sem):
        c = lax.axis_index("c")
        s = lax.axis_index("s")
        wid = s * NC + c

        sl = pl.ds(s * RPT, RPT)
        pltpu.sync_copy(g_hbm.at[sl], acc_sh.at[sl])

        @pl.when(s == 0)
        def _():
            tl = pl.ds(TAIL_OFF, TAIL)
            pltpu.sync_copy(g_hbm.at[tl], acc_sh.at[tl])

        plsc.subcore_barrier()

        def chunk_loop(ci, _):
            pltpu.sync_copy(src_hbm.at[wid, ci], src_v)
            pltpu.sync_copy(dst_hbm.at[wid, ci], dst_v)
            pltpu.async_copy(g_hbm.at[src_v.at[0]], rows_v.at[0], sem)

            def step(j, _):
                p = j % 2
                pltpu.make_async_copy(
                    g_hbm.at[src_v.at[j]], rows_v.at[p], sem).wait()

                @pl.when(j + 1 < CHUNK)
                def _():
                    pltpu.async_copy(
                        g_hbm.at[src_v.at[j + 1]], rows_v.at[1 - p], sem)

                pltpu.sync_copy(rows_v.at[p], acc_sh.at[dst_v.at[j]], add=True)
                return 0

            lax.fori_loop(0, CHUNK, step, 0)
            return 0

        lax.fori_loop(0, NCHUNK, chunk_loop, 0)
        plsc.subcore_barrier()

        pltpu.sync_copy(acc_sh.at[sl], out_hbm.at[c, sl])

        @pl.when(s == 0)
        def _():
            tl = pl.ds(TAIL_OFF, TAIL)
            pltpu.sync_copy(acc_sh.at[tl], out_hbm.at[c, tl])

    return agg


_agg128 = _make_agg_kernel(128)
_agg64 = _make_agg_kernel(64)


_RB = 1000
_GRID = N_NODES // _RB


def _dis_block(degp_ref):
    deg = degp_ref[0][:, 0:1] + degp_ref[1][:, 0:1] + 1.0
    return lax.rsqrt(deg)


def _tc1_body(degp_ref, x_ref, w1_ref, g1_ref):
    dis = _dis_block(degp_ref)
    g1_ref[...] = dis * jnp.dot(x_ref[...], w1_ref[...],
                                preferred_element_type=jnp.float32)


def _tc2_body(degp_ref, a1_ref, g1_ref, b1_ref, w2_ref, g2_ref):
    dis = _dis_block(degp_ref)
    h = a1_ref[0] + a1_ref[1] - g1_ref[...]
    h = jnp.maximum(dis * h + b1_ref[...], 0.0)
    g2_ref[...] = dis * jnp.dot(h, w2_ref[...],
                                preferred_element_type=jnp.float32)


def _tc3_body(degp_ref, a2_ref, g2_ref, b2_ref, out_ref):
    dis = _dis_block(degp_ref)
    o = a2_ref[0] + a2_ref[1] - g2_ref[...]
    out_ref[...] = jnp.maximum(dis * o + b2_ref[...], 0.0)


def _degp_spec():
    return pl.BlockSpec((NC, _RB, HIST_W), lambda i: (0, i, 0))


def _rows_spec(d):
    return pl.BlockSpec((_RB, d), lambda i: (i, 0))


def _parts_spec(d):
    return pl.BlockSpec((NC, _RB, d), lambda i: (0, i, 0))


def _full_spec(a, b):
    return pl.BlockSpec((a, b), lambda i: (0, 0))


def kernel(x, edge_index, W1, b1, W2, b2):
    ei = edge_index.astype(jnp.int32)
    pad_src = jnp.zeros((NW, PAD), jnp.int32)
    pad_dst = jnp.broadcast_to(
        N_NODES + (jnp.arange(PAD, dtype=jnp.int32) % NJUNK), (NW, PAD))
    src = jnp.concatenate([ei[0].reshape(NW, EPW), pad_src],
                          axis=1).reshape(NW, NCHUNK, CHUNK, K)
    dst = jnp.concatenate([ei[1].reshape(NW, EPW), pad_dst],
                          axis=1).reshape(NW, NCHUNK, CHUNK, K)

    degp = _deg_kernel(dst)

    g1 = pl.pallas_call(
        _tc1_body,
        grid=(_GRID,),
        in_specs=[_degp_spec(), _rows_spec(128), _full_spec(128, 128)],
        out_specs=_rows_spec(128),
        out_shape=jax.ShapeDtypeStruct((N_NODES, 128), jnp.float32),
    )(degp, x, W1)

    a1 = _agg128(g1, src, dst)

    g2 = pl.pallas_call(
        _tc2_body,
        grid=(_GRID,),
        in_specs=[_degp_spec(), _parts_spec(128), _rows_spec(128),
                  _full_spec(1, 128), _full_spec(128, 64)],
        out_specs=_rows_spec(64),
        out_shape=jax.ShapeDtypeStruct((N_NODES, 64), jnp.float32),
    )(degp, a1, g1, b1.reshape(1, 128), W2)

    a2 = _agg64(g2, src, dst)

    out = pl.pallas_call(
        _tc3_body,
        grid=(_GRID,),
        in_specs=[_degp_spec(), _parts_spec(64), _rows_spec(64),
                  _full_spec(1, 64)],
        out_specs=_rows_spec(64),
        out_shape=jax.ShapeDtypeStruct((N_NODES, 64), jnp.float32),
    )(degp, a2, g2, b2.reshape(1, 64))

    return out


# speedup vs baseline: 2.1110x; 1.0979x over previous
import functools

import jax
import jax.numpy as jnp
from jax import lax
from jax.experimental import pallas as pl
from jax.experimental.pallas import tpu as pltpu, tpu_sc as plsc

N_NODES = 10000
N_EDGES = 320000
NC, NS = 2, 16
NW = NC * NS
EPW = N_EDGES // NW
K = 80
CHUNK = 25
NCHUNK = 5
PAD = NCHUNK * CHUNK * K - EPW
NJUNK = 8
ACC_ROWS = N_NODES + NJUNK
RPT = 624
TAIL = N_NODES - NS * RPT
TAIL_OFF = NS * RPT
FB = 104
NFB = RPT // FB
HIST_W = 16

_MESH = plsc.VectorSubcoreMesh(core_axis_name="c", subcore_axis_name="s")


@functools.partial(
    pl.kernel,
    out_type=jax.ShapeDtypeStruct((NC, N_NODES, HIST_W), jnp.float32),
    mesh=_MESH,
    scratch_types=[
        pltpu.VMEM((CHUNK, K), jnp.int32),
        pltpu.VMEM((K, HIST_W), jnp.float32),
        pltpu.VMEM((FB, HIST_W), jnp.float32),
        pltpu.VMEM_SHARED((ACC_ROWS, HIST_W), jnp.float32),
    ],
    compiler_params=pltpu.CompilerParams(use_tc_tiling_on_sc=False),
)
def _deg_kernel(dst_hbm, out_hbm, dst_v, ones_v, stage_v, hist_sh):
    c = lax.axis_index("c")
    s = lax.axis_index("s")
    wid = s * NC + c

    def fill_stage(i, _):
        stage_v[i] = jnp.zeros((HIST_W,), jnp.float32)
        return 0

    lax.fori_loop(0, FB, fill_stage, 0)

    def fill_ones(i, _):
        ones_v[i] = jnp.ones((HIST_W,), jnp.float32)
        return 0

    lax.fori_loop(0, K, fill_ones, 0)

    def zero_block(k, _):
        pltpu.sync_copy(stage_v, hist_sh.at[pl.ds(s * RPT + k * FB, FB)])
        return 0

    lax.fori_loop(0, NFB, zero_block, 0)

    @pl.when(s == 0)
    def _():
        pltpu.sync_copy(stage_v.at[pl.ds(0, TAIL)],
                        hist_sh.at[pl.ds(TAIL_OFF, TAIL)])

    plsc.subcore_barrier()

    def chunk_loop(ci, _):
        pltpu.sync_copy(dst_hbm.at[wid, ci], dst_v)

        def step(j, _):
            pltpu.sync_copy(ones_v, hist_sh.at[dst_v.at[j]], add=True)
            return 0

        lax.fori_loop(0, CHUNK, step, 0)
        return 0

    lax.fori_loop(0, NCHUNK, chunk_loop, 0)
    plsc.subcore_barrier()

    def flush_block(k, _):
        pltpu.sync_copy(hist_sh.at[pl.ds(s * RPT + k * FB, FB)], stage_v)
        pltpu.sync_copy(stage_v, out_hbm.at[c, pl.ds(s * RPT + k * FB, FB)])
        return 0

    lax.fori_loop(0, NFB, flush_block, 0)

    @pl.when(s == 0)
    def _():
        pltpu.sync_copy(hist_sh.at[pl.ds(TAIL_OFF, TAIL)],
                        stage_v.at[pl.ds(0, TAIL)])
        pltpu.sync_copy(stage_v.at[pl.ds(0, TAIL)],
                        out_hbm.at[c, pl.ds(TAIL_OFF, TAIL)])


def _make_agg_kernel(d, table_in_spmem=False):
    scratch = [
        pltpu.VMEM((CHUNK, K), jnp.int32),
        pltpu.VMEM((CHUNK, K), jnp.int32),
        pltpu.VMEM((2, K, d), jnp.float32),
        pltpu.VMEM_SHARED((ACC_ROWS, d), jnp.float32),
    ]
    if table_in_spmem:
        scratch.append(pltpu.VMEM_SHARED((N_NODES, d), jnp.float32))
    scratch.append(pltpu.SemaphoreType.DMA)

    @functools.partial(
        pl.kernel,
        out_type=jax.ShapeDtypeStruct((NC, N_NODES, d), jnp.float32),
        mesh=_MESH,
        scratch_types=scratch,
        compiler_params=pltpu.CompilerParams(use_tc_tiling_on_sc=False),
    )
    def agg(g_hbm, src_hbm, dst_hbm, out_hbm,
            src_v, dst_v, rows_v, acc_sh, *rest):
        if table_in_spmem:
            tbl_sh, sem = rest
        else:
            (sem,) = rest
            tbl_sh = None
        gsrc = tbl_sh if table_in_spmem else g_hbm
        c = lax.axis_index("c")
        s = lax.axis_index("s")
        wid = s * NC + c

        sl = pl.ds(s * RPT, RPT)
        pltpu.sync_copy(g_hbm.at[sl], acc_sh.at[sl])
        if table_in_spmem:
            pltpu.sync_copy(g_hbm.at[sl], tbl_sh.at[sl])

        @pl.when(s == 0)
        def _():
            tl = pl.ds(TAIL_OFF, TAIL)
            pltpu.sync_copy(g_hbm.at[tl], acc_sh.at[tl])
            if table_in_spmem:
                pltpu.sync_copy(g_hbm.at[tl], tbl_sh.at[tl])

        plsc.subcore_barrier()

        def chunk_loop(ci, _):
            pltpu.sync_copy(src_hbm.at[wid, ci], src_v)
            pltpu.sync_copy(dst_hbm.at[wid, ci], dst_v)
            pltpu.async_copy(gsrc.at[src_v.at[0]], rows_v.at[0], sem)

            def step(j, _):
                p = j % 2
                pltpu.make_async_copy(
                    gsrc.at[src_v.at[j]], rows_v.at[p], sem).wait()

                @pl.when(j + 1 < CHUNK)
                def _():
                    pltpu.async_copy(
                        gsrc.at[src_v.at[j + 1]], rows_v.at[1 - p], sem)

                pltpu.sync_copy(rows_v.at[p], acc_sh.at[dst_v.at[j]], add=True)
                return 0

            lax.fori_loop(0, CHUNK, step, 0)
            return 0

        lax.fori_loop(0, NCHUNK, chunk_loop, 0)
        plsc.subcore_barrier()

        pltpu.sync_copy(acc_sh.at[sl], out_hbm.at[c, sl])

        @pl.when(s == 0)
        def _():
            tl = pl.ds(TAIL_OFF, TAIL)
            pltpu.sync_copy(acc_sh.at[tl], out_hbm.at[c, tl])

    return agg


_agg128 = _make_agg_kernel(128)
_agg64 = _make_agg_kernel(64, table_in_spmem=True)


_RB = 1000
_GRID = N_NODES // _RB


def _dis_block(degp_ref):
    deg = degp_ref[0][:, 0:1] + degp_ref[1][:, 0:1] + 1.0
    return lax.rsqrt(deg)


def _tc1_body(degp_ref, x_ref, w1_ref, g1_ref):
    dis = _dis_block(degp_ref)
    g1_ref[...] = dis * jnp.dot(x_ref[...], w1_ref[...],
                                preferred_element_type=jnp.float32)


def _tc2_body(degp_ref, a1_ref, g1_ref, b1_ref, w2_ref, g2_ref):
    dis = _dis_block(degp_ref)
    h = a1_ref[0] + a1_ref[1] - g1_ref[...]
    h = jnp.maximum(dis * h + b1_ref[...], 0.0)
    g2_ref[...] = dis * jnp.dot(h, w2_ref[...],
                                preferred_element_type=jnp.float32)


def _tc3_body(degp_ref, a2_ref, g2_ref, b2_ref, out_ref):
    dis = _dis_block(degp_ref)
    o = a2_ref[0] + a2_ref[1] - g2_ref[...]
    out_ref[...] = jnp.maximum(dis * o + b2_ref[...], 0.0)


def _degp_spec():
    return pl.BlockSpec((NC, _RB, HIST_W), lambda i: (0, i, 0))


def _rows_spec(d):
    return pl.BlockSpec((_RB, d), lambda i: (i, 0))


def _parts_spec(d):
    return pl.BlockSpec((NC, _RB, d), lambda i: (0, i, 0))


def _full_spec(a, b):
    return pl.BlockSpec((a, b), lambda i: (0, 0))


def kernel(x, edge_index, W1, b1, W2, b2):
    ei = edge_index.astype(jnp.int32)
    pad_src = jnp.zeros((NW, PAD), jnp.int32)
    pad_dst = jnp.broadcast_to(
        N_NODES + (jnp.arange(PAD, dtype=jnp.int32) % NJUNK), (NW, PAD))
    src = jnp.concatenate([ei[0].reshape(NW, EPW), pad_src],
                          axis=1).reshape(NW, NCHUNK, CHUNK, K)
    dst = jnp.concatenate([ei[1].reshape(NW, EPW), pad_dst],
                          axis=1).reshape(NW, NCHUNK, CHUNK, K)

    degp = _deg_kernel(dst)

    g1 = pl.pallas_call(
        _tc1_body,
        grid=(_GRID,),
        in_specs=[_degp_spec(), _rows_spec(128), _full_spec(128, 128)],
        out_specs=_rows_spec(128),
        out_shape=jax.ShapeDtypeStruct((N_NODES, 128), jnp.float32),
    )(degp, x, W1)

    a1 = _agg128(g1, src, dst)

    g2 = pl.pallas_call(
        _tc2_body,
        grid=(_GRID,),
        in_specs=[_degp_spec(), _parts_spec(128), _rows_spec(128),
                  _full_spec(1, 128), _full_spec(128, 64)],
        out_specs=_rows_spec(64),
        out_shape=jax.ShapeDtypeStruct((N_NODES, 64), jnp.float32),
    )(degp, a1, g1, b1.reshape(1, 128), W2)

    a2 = _agg64(g2, src, dst)

    out = pl.pallas_call(
        _tc3_body,
        grid=(_GRID,),
        in_specs=[_degp_spec(), _parts_spec(64), _rows_spec(64),
                  _full_spec(1, 64)],
        out_specs=_rows_spec(64),
        out_shape=jax.ShapeDtypeStruct((N_NODES, 64), jnp.float32),
    )(degp, a2, g2, b2.reshape(1, 64))

    return out


# speedup vs baseline: 2.5978x; 1.2306x over previous
import functools

import jax
import jax.numpy as jnp
from jax import lax
from jax.experimental import pallas as pl
from jax.experimental.pallas import tpu as pltpu, tpu_sc as plsc

N_NODES = 10000
N_EDGES = 320000
NC, NS = 2, 16
NW = NC * NS
EPW = N_EDGES // NW
K = 80
CHUNK = 25
NCHUNK = 5
PAD = NCHUNK * CHUNK * K - EPW
NJUNK = 8
ACC_ROWS = N_NODES + NJUNK
RPT = 624
TAIL = N_NODES - NS * RPT
TAIL_OFF = NS * RPT
FB = 104
NFB = RPT // FB
HIST_W = 16

_MESH = plsc.VectorSubcoreMesh(core_axis_name="c", subcore_axis_name="s")


@functools.partial(
    pl.kernel,
    out_type=jax.ShapeDtypeStruct((NC, N_NODES, HIST_W), jnp.float32),
    mesh=_MESH,
    scratch_types=[
        pltpu.VMEM((CHUNK, K), jnp.int32),
        pltpu.VMEM((K, HIST_W), jnp.float32),
        pltpu.VMEM((FB, HIST_W), jnp.float32),
        pltpu.VMEM_SHARED((ACC_ROWS, HIST_W), jnp.float32),
    ],
    compiler_params=pltpu.CompilerParams(use_tc_tiling_on_sc=False),
)
def _deg_kernel(dst_hbm, out_hbm, dst_v, ones_v, stage_v, hist_sh):
    c = lax.axis_index("c")
    s = lax.axis_index("s")
    wid = s * NC + c

    def fill_stage(i, _):
        stage_v[i] = jnp.zeros((HIST_W,), jnp.float32)
        return 0

    lax.fori_loop(0, FB, fill_stage, 0)

    def fill_ones(i, _):
        ones_v[i] = jnp.ones((HIST_W,), jnp.float32)
        return 0

    lax.fori_loop(0, K, fill_ones, 0)

    def zero_block(k, _):
        pltpu.sync_copy(stage_v, hist_sh.at[pl.ds(s * RPT + k * FB, FB)])
        return 0

    lax.fori_loop(0, NFB, zero_block, 0)

    @pl.when(s == 0)
    def _():
        pltpu.sync_copy(stage_v.at[pl.ds(0, TAIL)],
                        hist_sh.at[pl.ds(TAIL_OFF, TAIL)])

    plsc.subcore_barrier()

    def chunk_loop(ci, _):
        pltpu.sync_copy(dst_hbm.at[wid, ci], dst_v)

        def step(j, _):
            pltpu.sync_copy(ones_v, hist_sh.at[dst_v.at[j]], add=True)
            return 0

        lax.fori_loop(0, CHUNK, step, 0)
        return 0

    lax.fori_loop(0, NCHUNK, chunk_loop, 0)
    plsc.subcore_barrier()

    def flush_block(k, _):
        pltpu.sync_copy(hist_sh.at[pl.ds(s * RPT + k * FB, FB)], stage_v)
        pltpu.sync_copy(stage_v, out_hbm.at[c, pl.ds(s * RPT + k * FB, FB)])
        return 0

    lax.fori_loop(0, NFB, flush_block, 0)

    @pl.when(s == 0)
    def _():
        pltpu.sync_copy(hist_sh.at[pl.ds(TAIL_OFF, TAIL)],
                        stage_v.at[pl.ds(0, TAIL)])
        pltpu.sync_copy(stage_v.at[pl.ds(0, TAIL)],
                        out_hbm.at[c, pl.ds(TAIL_OFF, TAIL)])


def _make_agg_kernel(d, table_in_spmem=False):
    scratch = [
        pltpu.VMEM((CHUNK, K), jnp.int32),
        pltpu.VMEM((CHUNK, K), jnp.int32),
        pltpu.VMEM((3, K, d), jnp.float32),
        pltpu.VMEM_SHARED((ACC_ROWS, d), jnp.float32),
    ]
    if table_in_spmem:
        scratch.append(pltpu.VMEM_SHARED((N_NODES, d), jnp.float32))
    scratch.append(pltpu.SemaphoreType.DMA)
    scratch.append(pltpu.SemaphoreType.DMA)

    @functools.partial(
        pl.kernel,
        out_type=jax.ShapeDtypeStruct((NC, N_NODES, d), jnp.float32),
        mesh=_MESH,
        scratch_types=scratch,
        compiler_params=pltpu.CompilerParams(use_tc_tiling_on_sc=False),
    )
    def agg(g_hbm, src_hbm, dst_hbm, out_hbm,
            src_v, dst_v, rows_v, acc_sh, *rest):
        if table_in_spmem:
            tbl_sh, sem, sem_s = rest
        else:
            sem, sem_s = rest
            tbl_sh = None
        gsrc = tbl_sh if table_in_spmem else g_hbm
        c = lax.axis_index("c")
        s = lax.axis_index("s")
        wid = s * NC + c

        sl = pl.ds(s * RPT, RPT)
        pltpu.sync_copy(g_hbm.at[sl], acc_sh.at[sl])
        if table_in_spmem:
            pltpu.sync_copy(g_hbm.at[sl], tbl_sh.at[sl])

        @pl.when(s == 0)
        def _():
            tl = pl.ds(TAIL_OFF, TAIL)
            pltpu.sync_copy(g_hbm.at[tl], acc_sh.at[tl])
            if table_in_spmem:
                pltpu.sync_copy(g_hbm.at[tl], tbl_sh.at[tl])

        plsc.subcore_barrier()

        def chunk_loop(ci, _):
            pltpu.sync_copy(src_hbm.at[wid, ci], src_v)
            pltpu.sync_copy(dst_hbm.at[wid, ci], dst_v)
            pltpu.async_copy(gsrc.at[src_v.at[0]], rows_v.at[0], sem)
            pltpu.async_copy(gsrc.at[src_v.at[1]], rows_v.at[1], sem)

            def step(j, _):
                p = j % 3
                pltpu.make_async_copy(
                    gsrc.at[src_v.at[j]], rows_v.at[p], sem).wait()
                pltpu.async_copy(
                    rows_v.at[p], acc_sh.at[dst_v.at[j]], sem_s, add=True)

                @pl.when(j + 2 < CHUNK)
                def _():
                    @pl.when(j >= 1)
                    def _():
                        pltpu.make_async_copy(
                            rows_v.at[(j + 2) % 3],
                            acc_sh.at[dst_v.at[j - 1]], sem_s).wait()

                    pltpu.async_copy(
                        gsrc.at[src_v.at[j + 2]], rows_v.at[(j + 2) % 3], sem)

                return 0

            lax.fori_loop(0, CHUNK, step, 0)

            def drain(i, _):
                pltpu.make_async_copy(
                    rows_v.at[0], acc_sh.at[dst_v.at[0]], sem_s).wait()
                return 0

            lax.fori_loop(0, 3, drain, 0)
            return 0

        lax.fori_loop(0, NCHUNK, chunk_loop, 0)
        plsc.subcore_barrier()

        pltpu.sync_copy(acc_sh.at[sl], out_hbm.at[c, sl])

        @pl.when(s == 0)
        def _():
            tl = pl.ds(TAIL_OFF, TAIL)
            pltpu.sync_copy(acc_sh.at[tl], out_hbm.at[c, tl])

    return agg


_agg128 = _make_agg_kernel(128)
_agg64 = _make_agg_kernel(64, table_in_spmem=True)


_RB = 1000
_GRID = N_NODES // _RB


def _dis_block(degp_ref):
    deg = degp_ref[0][:, 0:1] + degp_ref[1][:, 0:1] + 1.0
    return lax.rsqrt(deg)


def _tc1_body(degp_ref, x_ref, w1_ref, g1_ref):
    dis = _dis_block(degp_ref)
    g1_ref[...] = dis * jnp.dot(x_ref[...], w1_ref[...],
                                preferred_element_type=jnp.float32)


def _tc2_body(degp_ref, a1_ref, g1_ref, b1_ref, w2_ref, g2_ref):
    dis = _dis_block(degp_ref)
    h = a1_ref[0] + a1_ref[1] - g1_ref[...]
    h = jnp.maximum(dis * h + b1_ref[...], 0.0)
    g2_ref[...] = dis * jnp.dot(h, w2_ref[...],
                                preferred_element_type=jnp.float32)


def _tc3_body(degp_ref, a2_ref, g2_ref, b2_ref, out_ref):
    dis = _dis_block(degp_ref)
    o = a2_ref[0] + a2_ref[1] - g2_ref[...]
    out_ref[...] = jnp.maximum(dis * o + b2_ref[...], 0.0)


def _degp_spec():
    return pl.BlockSpec((NC, _RB, HIST_W), lambda i: (0, i, 0))


def _rows_spec(d):
    return pl.BlockSpec((_RB, d), lambda i: (i, 0))


def _parts_spec(d):
    return pl.BlockSpec((NC, _RB, d), lambda i: (0, i, 0))


def _full_spec(a, b):
    return pl.BlockSpec((a, b), lambda i: (0, 0))


def kernel(x, edge_index, W1, b1, W2, b2):
    ei = edge_index.astype(jnp.int32)
    pad_src = jnp.zeros((NW, PAD), jnp.int32)
    pad_dst = jnp.broadcast_to(
        N_NODES + (jnp.arange(PAD, dtype=jnp.int32) % NJUNK), (NW, PAD))
    src = jnp.concatenate([ei[0].reshape(NW, EPW), pad_src],
                          axis=1).reshape(NW, NCHUNK, CHUNK, K)
    dst = jnp.concatenate([ei[1].reshape(NW, EPW), pad_dst],
                          axis=1).reshape(NW, NCHUNK, CHUNK, K)

    degp = _deg_kernel(dst)

    g1 = pl.pallas_call(
        _tc1_body,
        grid=(_GRID,),
        in_specs=[_degp_spec(), _rows_spec(128), _full_spec(128, 128)],
        out_specs=_rows_spec(128),
        out_shape=jax.ShapeDtypeStruct((N_NODES, 128), jnp.float32),
    )(degp, x, W1)

    a1 = _agg128(g1, src, dst)

    g2 = pl.pallas_call(
        _tc2_body,
        grid=(_GRID,),
        in_specs=[_degp_spec(), _parts_spec(128), _rows_spec(128),
                  _full_spec(1, 128), _full_spec(128, 64)],
        out_specs=_rows_spec(64),
        out_shape=jax.ShapeDtypeStruct((N_NODES, 64), jnp.float32),
    )(degp, a1, g1, b1.reshape(1, 128), W2)

    a2 = _agg64(g2, src, dst)

    out = pl.pallas_call(
        _tc3_body,
        grid=(_GRID,),
        in_specs=[_degp_spec(), _parts_spec(64), _rows_spec(64),
                  _full_spec(1, 64)],
        out_specs=_rows_spec(64),
        out_shape=jax.ShapeDtypeStruct((N_NODES, 64), jnp.float32),
    )(degp, a2, g2, b2.reshape(1, 64))

    return out


# speedup vs baseline: 2.6250x; 1.0105x over previous
import functools

import jax
import jax.numpy as jnp
from jax import lax
from jax.experimental import pallas as pl
from jax.experimental.pallas import tpu as pltpu, tpu_sc as plsc

N_NODES = 10000
N_EDGES = 320000
NC, NS = 2, 16
NW = NC * NS
EPW = N_EDGES // NW
K = 80
CHUNK = 25
NCHUNK = 5
PAD = NCHUNK * CHUNK * K - EPW
NJUNK = 8
ACC_ROWS = N_NODES + NJUNK
RPT = 624
TAIL = N_NODES - NS * RPT
TAIL_OFF = NS * RPT
FB = 104
NFB = RPT // FB
HIST_W = 16

_MESH = plsc.VectorSubcoreMesh(core_axis_name="c", subcore_axis_name="s")


@functools.partial(
    pl.kernel,
    out_type=jax.ShapeDtypeStruct((NC, N_NODES, HIST_W), jnp.float32),
    mesh=_MESH,
    scratch_types=[
        pltpu.VMEM((CHUNK, K), jnp.int32),
        pltpu.VMEM((K, HIST_W), jnp.float32),
        pltpu.VMEM((FB, HIST_W), jnp.float32),
        pltpu.VMEM_SHARED((ACC_ROWS, HIST_W), jnp.float32),
        pltpu.SemaphoreType.DMA,
    ],
    compiler_params=pltpu.CompilerParams(use_tc_tiling_on_sc=False),
)
def _deg_kernel(dst_hbm, out_hbm, dst_v, ones_v, stage_v, hist_sh, sem):
    c = lax.axis_index("c")
    s = lax.axis_index("s")
    wid = s * NC + c

    def fill_stage(i, _):
        stage_v[i] = jnp.zeros((HIST_W,), jnp.float32)
        return 0

    lax.fori_loop(0, FB, fill_stage, 0)

    def fill_ones(i, _):
        ones_v[i] = jnp.ones((HIST_W,), jnp.float32)
        return 0

    lax.fori_loop(0, K, fill_ones, 0)

    def zero_block(k, _):
        pltpu.sync_copy(stage_v, hist_sh.at[pl.ds(s * RPT + k * FB, FB)])
        return 0

    lax.fori_loop(0, NFB, zero_block, 0)

    @pl.when(s == 0)
    def _():
        pltpu.sync_copy(stage_v.at[pl.ds(0, TAIL)],
                        hist_sh.at[pl.ds(TAIL_OFF, TAIL)])

    plsc.subcore_barrier()

    def chunk_loop(ci, _):
        pltpu.sync_copy(dst_hbm.at[wid, ci], dst_v)

        def step(j, _):
            pltpu.async_copy(ones_v, hist_sh.at[dst_v.at[j]], sem, add=True)
            return 0

        lax.fori_loop(0, CHUNK, step, 0)

        def drain(j, _):
            pltpu.make_async_copy(ones_v, hist_sh.at[dst_v.at[0]], sem).wait()
            return 0

        lax.fori_loop(0, CHUNK, drain, 0)
        return 0

    lax.fori_loop(0, NCHUNK, chunk_loop, 0)
    plsc.subcore_barrier()

    def flush_block(k, _):
        pltpu.sync_copy(hist_sh.at[pl.ds(s * RPT + k * FB, FB)], stage_v)
        pltpu.sync_copy(stage_v, out_hbm.at[c, pl.ds(s * RPT + k * FB, FB)])
        return 0

    lax.fori_loop(0, NFB, flush_block, 0)

    @pl.when(s == 0)
    def _():
        pltpu.sync_copy(hist_sh.at[pl.ds(TAIL_OFF, TAIL)],
                        stage_v.at[pl.ds(0, TAIL)])
        pltpu.sync_copy(stage_v.at[pl.ds(0, TAIL)],
                        out_hbm.at[c, pl.ds(TAIL_OFF, TAIL)])


def _make_agg_kernel(d, table_in_spmem=False):
    scratch = [
        pltpu.VMEM((CHUNK, K), jnp.int32),
        pltpu.VMEM((CHUNK, K), jnp.int32),
        pltpu.VMEM((3, K, d), jnp.float32),
        pltpu.VMEM_SHARED((ACC_ROWS, d), jnp.float32),
    ]
    if table_in_spmem:
        scratch.append(pltpu.VMEM_SHARED((N_NODES, d), jnp.float32))
    scratch.append(pltpu.SemaphoreType.DMA)
    scratch.append(pltpu.SemaphoreType.DMA)

    @functools.partial(
        pl.kernel,
        out_type=jax.ShapeDtypeStruct((NC, N_NODES, d), jnp.float32),
        mesh=_MESH,
        scratch_types=scratch,
        compiler_params=pltpu.CompilerParams(use_tc_tiling_on_sc=False),
    )
    def agg(g_hbm, src_hbm, dst_hbm, out_hbm,
            src_v, dst_v, rows_v, acc_sh, *rest):
        if table_in_spmem:
            tbl_sh, sem, sem_s = rest
        else:
            sem, sem_s = rest
            tbl_sh = None
        gsrc = tbl_sh if table_in_spmem else g_hbm
        c = lax.axis_index("c")
        s = lax.axis_index("s")
        wid = s * NC + c

        sl = pl.ds(s * RPT, RPT)
        pltpu.sync_copy(g_hbm.at[sl], acc_sh.at[sl])
        if table_in_spmem:
            pltpu.sync_copy(g_hbm.at[sl], tbl_sh.at[sl])

        @pl.when(s == 0)
        def _():
            tl = pl.ds(TAIL_OFF, TAIL)
            pltpu.sync_copy(g_hbm.at[tl], acc_sh.at[tl])
            if table_in_spmem:
                pltpu.sync_copy(g_hbm.at[tl], tbl_sh.at[tl])

        plsc.subcore_barrier()

        def chunk_loop(ci, _):
            pltpu.sync_copy(src_hbm.at[wid, ci], src_v)
            pltpu.sync_copy(dst_hbm.at[wid, ci], dst_v)
            pltpu.async_copy(gsrc.at[src_v.at[0]], rows_v.at[0], sem)
            pltpu.async_copy(gsrc.at[src_v.at[1]], rows_v.at[1], sem)

            def step(j, _):
                p = j % 3
                pltpu.make_async_copy(
                    gsrc.at[src_v.at[j]], rows_v.at[p], sem).wait()
                pltpu.async_copy(
                    rows_v.at[p], acc_sh.at[dst_v.at[j]], sem_s, add=True)

                @pl.when(j + 2 < CHUNK)
                def _():
                    @pl.when(j >= 1)
                    def _():
                        pltpu.make_async_copy(
                            rows_v.at[(j + 2) % 3],
                            acc_sh.at[dst_v.at[j - 1]], sem_s).wait()

                    pltpu.async_copy(
                        gsrc.at[src_v.at[j + 2]], rows_v.at[(j + 2) % 3], sem)

                return 0

            lax.fori_loop(0, CHUNK, step, 0)

            def drain(i, _):
                pltpu.make_async_copy(
                    rows_v.at[0], acc_sh.at[dst_v.at[0]], sem_s).wait()
                return 0

            lax.fori_loop(0, 3, drain, 0)
            return 0

        lax.fori_loop(0, NCHUNK, chunk_loop, 0)
        plsc.subcore_barrier()

        pltpu.sync_copy(acc_sh.at[sl], out_hbm.at[c, sl])

        @pl.when(s == 0)
        def _():
            tl = pl.ds(TAIL_OFF, TAIL)
            pltpu.sync_copy(acc_sh.at[tl], out_hbm.at[c, tl])

    return agg


_agg128 = _make_agg_kernel(128)
_agg64 = _make_agg_kernel(64)


_RB = 1000
_GRID = N_NODES // _RB


def _dis_block(degp_ref):
    deg = degp_ref[0][:, 0:1] + degp_ref[1][:, 0:1] + 1.0
    return lax.rsqrt(deg)


def _tc1_body(degp_ref, x_ref, w1_ref, g1_ref):
    dis = _dis_block(degp_ref)
    g1_ref[...] = dis * jnp.dot(x_ref[...], w1_ref[...],
                                preferred_element_type=jnp.float32)


def _tc2_body(degp_ref, a1_ref, g1_ref, b1_ref, w2_ref, g2_ref):
    dis = _dis_block(degp_ref)
    h = a1_ref[0] + a1_ref[1] - g1_ref[...]
    h = jnp.maximum(dis * h + b1_ref[...], 0.0)
    g2_ref[...] = dis * jnp.dot(h, w2_ref[...],
                                preferred_element_type=jnp.float32)


def _tc3_body(degp_ref, a2_ref, g2_ref, b2_ref, out_ref):
    dis = _dis_block(degp_ref)
    o = a2_ref[0] + a2_ref[1] - g2_ref[...]
    out_ref[...] = jnp.maximum(dis * o + b2_ref[...], 0.0)


def _degp_spec():
    return pl.BlockSpec((NC, _RB, HIST_W), lambda i: (0, i, 0))


def _rows_spec(d):
    return pl.BlockSpec((_RB, d), lambda i: (i, 0))


def _parts_spec(d):
    return pl.BlockSpec((NC, _RB, d), lambda i: (0, i, 0))


def _full_spec(a, b):
    return pl.BlockSpec((a, b), lambda i: (0, 0))


def kernel(x, edge_index, W1, b1, W2, b2):
    ei = edge_index.astype(jnp.int32)
    pad_src = jnp.zeros((NW, PAD), jnp.int32)
    pad_dst = jnp.broadcast_to(
        N_NODES + (jnp.arange(PAD, dtype=jnp.int32) % NJUNK), (NW, PAD))
    src = jnp.concatenate([ei[0].reshape(NW, EPW), pad_src],
                          axis=1).reshape(NW, NCHUNK, CHUNK, K)
    dst = jnp.concatenate([ei[1].reshape(NW, EPW), pad_dst],
                          axis=1).reshape(NW, NCHUNK, CHUNK, K)

    degp = _deg_kernel(dst)

    g1 = pl.pallas_call(
        _tc1_body,
        grid=(_GRID,),
        in_specs=[_degp_spec(), _rows_spec(128), _full_spec(128, 128)],
        out_specs=_rows_spec(128),
        out_shape=jax.ShapeDtypeStruct((N_NODES, 128), jnp.float32),
    )(degp, x, W1)

    a1 = _agg128(g1, src, dst)

    g2 = pl.pallas_call(
        _tc2_body,
        grid=(_GRID,),
        in_specs=[_degp_spec(), _parts_spec(128), _rows_spec(128),
                  _full_spec(1, 128), _full_spec(128, 64)],
        out_specs=_rows_spec(64),
        out_shape=jax.ShapeDtypeStruct((N_NODES, 64), jnp.float32),
    )(degp, a1, g1, b1.reshape(1, 128), W2)

    a2 = _agg64(g2, src, dst)

    out = pl.pallas_call(
        _tc3_body,
        grid=(_GRID,),
        in_specs=[_degp_spec(), _parts_spec(64), _rows_spec(64),
                  _full_spec(1, 64)],
        out_specs=_rows_spec(64),
        out_shape=jax.ShapeDtypeStruct((N_NODES, 64), jnp.float32),
    )(degp, a2, g2, b2.reshape(1, 64))

    return out
